# baseline (speedup 1.0000x reference)
import functools

import jax
import jax.numpy as jnp
from jax import lax
from jax.experimental import pallas as pl
from jax.experimental.pallas import tpu as pltpu
from jax.experimental.pallas import tpu_sc as plsc

_N = 10000
_E = 160000
_D = 256
_H = 128
_C = 2

_NP = 10240
_NC, _NS = 2, 16
_NW = _NC * _NS
_EP = 163840
_EPW = _EP // _NW
_CH = 128
_NCHUNK = _EPW // _CH
_RPS = _NP // _NS
_NBUF = 2
_HR = _NP // 2

_mesh = plsc.VectorSubcoreMesh(core_axis_name="c", subcore_axis_name="s")


def _deg_body(col_hbm, zero_hbm, out_hbm, colbuf, lhist, cntbuf):
    c = lax.axis_index("c")
    s = lax.axis_index("s")
    wid = c * _NS + s
    pltpu.sync_copy(col_hbm.at[pl.ds(wid * _EPW, _EPW)], colbuf)
    iota = lax.iota(jnp.int32, 16)
    laneoff = iota * _HR
    ones16 = jnp.ones((16,), jnp.float32)

    for p in range(_NP // _HR):
        pltpu.sync_copy(zero_hbm, lhist)

        def _scan(i, carry):
            idx = colbuf[pl.ds(i * 16, 16)]
            rel = idx - p * _HR
            m = (rel >= 0) & (rel < _HR)
            relc = jnp.where(m, rel, 0)
            addr = laneoff + relc
            cur = plsc.load_gather(lhist, [addr], mask=m)
            plsc.store_scatter(lhist, [addr], cur + ones16, mask=m)
            return carry

        lax.fori_loop(0, _EPW // 16, _scan, 0)

        def _reduce(k, carry):
            acc = lhist[pl.ds(k * 16, 16)]
            for t in range(1, 16):
                acc = acc + lhist[pl.ds(t * _HR + k * 16, 16)]
            cntbuf[pl.ds(k * 16, 16)] = acc
            return carry

        lax.fori_loop(0, _HR // 16, _reduce, 0)
        pltpu.sync_copy(cntbuf, out_hbm.at[wid, pl.ds(p * _HR, _HR)])


_deg_call = functools.partial(
    pl.kernel,
    out_type=jax.ShapeDtypeStruct((_NW, _NP), jnp.float32),
    mesh=_mesh,
    scratch_types=[
        pltpu.VMEM((_EPW,), jnp.int32),
        pltpu.VMEM((16 * _HR,), jnp.float32),
        pltpu.VMEM((_HR,), jnp.float32),
    ],
    compiler_params=pltpu.CompilerParams(needs_layout_passes=False),
)(_deg_body)


_NPH = 10
_PHR = _NP // _NPH
_CAP = _EPW // _CH + 1


def _scat_body(hp_hbm, row_hbm, col_hbm, zero_hbm, out_hbm,
               rawrow, rawcol, crow2, ccol2, rstage, cstage, gbuf, hpbuf,
               accum, gsem):
    c = lax.axis_index("c")
    s = lax.axis_index("s")
    wid = c * _NS + s
    pltpu.sync_copy(zero_hbm, accum.at[pl.ds(s * _RPS, _RPS)])
    pltpu.sync_copy(row_hbm.at[pl.ds(wid * _EPW, _EPW)], rawrow)
    pltpu.sync_copy(col_hbm.at[pl.ds(wid * _EPW, _EPW)], rawcol)
    plsc.subcore_barrier()

    zeros16i = jnp.zeros((16,), jnp.int32)
    dump16 = jnp.full((16,), _N, jnp.int32)

    def _flush(cc):
        for st, dst in ((rstage, crow2), (cstage, ccol2)):
            for k in range(8):
                dst[cc, pl.ds(k * 16, 16)] = st[pl.ds(k * 16, 16)]
            rem = st[pl.ds(128, 16)]
            st[pl.ds(0, 16)] = rem

    for p in range(_NPH):
        pltpu.sync_copy(
            hp_hbm.at[pl.ds(p * _PHR + s * (_PHR // _NS), _PHR // _NS)],
            hpbuf.at[pl.ds(s * (_PHR // _NS), _PHR // _NS)])
        plsc.subcore_barrier()

        def _scan(i, carry):
            wptr, cc = carry
            r = rawrow[pl.ds(i * 16, 16)]
            cl = rawcol[pl.ds(i * 16, 16)]
            rel = r - p * _PHR
            m = (rel >= 0) & (rel < _PHR)
            plsc.store_compressed(rstage.at[pl.ds(wptr, 16)], rel, mask=m)
            plsc.store_compressed(cstage.at[pl.ds(wptr, 16)], cl, mask=m)
            npop = plsc.all_reduce_population_count(m)
            wnew = wptr + npop[0]
            do_flush = wnew >= _CH

            @pl.when(do_flush)
            def _():
                _flush(cc)

            wptr2 = jnp.where(do_flush, wnew - _CH, wnew)
            cc2 = jnp.where(do_flush, cc + 1, cc)
            return (wptr2, cc2)

        wptr, cc = lax.fori_loop(0, _EPW // 16, _scan, (0, 0))

        for k in range(8):
            rstage[pl.ds(wptr + k * 16, 16)] = zeros16i
            cstage[pl.ds(wptr + k * 16, 16)] = dump16
        _flush(cc)
        trips = cc + 1

        def _stream(j, carry):
            pltpu.async_copy(hpbuf.at[crow2.at[j]], gbuf, gsem).wait()
            pltpu.sync_copy(gbuf, accum.at[ccol2.at[j]], add=True)
            return carry

        lax.fori_loop(0, trips, _stream, 0)
        plsc.subcore_barrier()

    pltpu.sync_copy(accum.at[pl.ds(s * _RPS, _RPS)],
                    out_hbm.at[c, pl.ds(s * _RPS, _RPS)])


_scat_call = functools.partial(
    pl.kernel,
    out_type=jax.ShapeDtypeStruct((_NC, _NP, _H), jnp.float32),
    mesh=_mesh,
    scratch_types=[
        pltpu.VMEM((_EPW,), jnp.int32),
        pltpu.VMEM((_EPW,), jnp.int32),
        pltpu.VMEM((_CAP, _CH), jnp.int32),
        pltpu.VMEM((_CAP, _CH), jnp.int32),
        pltpu.VMEM((_CH + 144,), jnp.int32),
        pltpu.VMEM((_CH + 144,), jnp.int32),
        pltpu.VMEM((_CH, _H), jnp.float32),
        pltpu.VMEM_SHARED((_PHR, _H), jnp.float32),
        pltpu.VMEM_SHARED((_NP, _H), jnp.float32),
        pltpu.SemaphoreType.DMA,
    ],
    compiler_params=pltpu.CompilerParams(needs_layout_passes=False),
)(_scat_body)


def _dis_block(degp_blk):
    deg = jnp.sum(degp_blk, axis=0) + 1.0
    dis = jax.lax.rsqrt(deg)
    return jax.lax.broadcast_in_dim(dis, (degp_blk.shape[1], _H), (0,))


def _mm_body(x_ref, w1_ref, degp_ref, hp_ref):
    dism = _dis_block(degp_ref[...])
    h = jnp.dot(x_ref[...], w1_ref[...], preferred_element_type=jnp.float32)
    hp_ref[...] = h * dism


_BM = 256


def _mm_call(x_p, W1, degp):
    grid = (_NP // _BM,)
    return pl.pallas_call(
        _mm_body,
        grid=grid,
        in_specs=[
            pl.BlockSpec((_BM, _D), lambda i: (i, 0)),
            pl.BlockSpec((_D, _H), lambda i: (0, 0)),
            pl.BlockSpec((_NW, _BM), lambda i: (0, i)),
        ],
        out_specs=pl.BlockSpec((_BM, _H), lambda i: (i, 0)),
        out_shape=jax.ShapeDtypeStruct((_NP, _H), jnp.float32),
    )(x_p, W1, degp)


def _tail_body(p_ref, hp_ref, degp_ref, b1_ref, w2_ref, b2_ref, out_ref):
    dism = _dis_block(degp_ref[...])
    sums = p_ref[0] + p_ref[1] + hp_ref[...]
    pre = sums * dism + b1_ref[...]
    act = jnp.maximum(pre, 0.0)
    out_ref[...] = jnp.dot(act, w2_ref[...],
                           preferred_element_type=jnp.float32) + b2_ref[...]


def _tail_call(partials, hp, degp, b1r, W2p, b2p):
    grid = (_NP // _BM,)
    return pl.pallas_call(
        _tail_body,
        grid=grid,
        in_specs=[
            pl.BlockSpec((_NC, _BM, _H), lambda i: (0, i, 0)),
            pl.BlockSpec((_BM, _H), lambda i: (i, 0)),
            pl.BlockSpec((_NW, _BM), lambda i: (0, i)),
            pl.BlockSpec((1, _H), lambda i: (0, 0)),
            pl.BlockSpec((_H, 8), lambda i: (0, 0)),
            pl.BlockSpec((1, 8), lambda i: (0, 0)),
        ],
        out_specs=pl.BlockSpec((_BM, 8), lambda i: (i, 0)),
        out_shape=jax.ShapeDtypeStruct((_NP, 8), jnp.float32),
    )(partials, hp, degp, b1r, W2p, b2p)


def kernel(x, edge_index, W1, b1, W2, b2):
    row = edge_index[0]
    col = edge_index[1]
    pad = _EP - _E
    rowp = jnp.concatenate([row, jnp.zeros((pad,), jnp.int32)])
    colp = jnp.concatenate([col, jnp.full((pad,), _N, jnp.int32)])
    row2 = rowp.reshape(_EP // _CH, _CH)
    col2 = colp.reshape(_EP // _CH, _CH)
    x_p = jnp.concatenate([x, jnp.zeros((_NP - _N, _D), jnp.float32)])
    zero_blk = jnp.zeros((_RPS, _H), jnp.float32)
    zero_hist = jnp.zeros((16 * _HR,), jnp.float32)
    b1r = b1.reshape(1, _H)
    W2p = jnp.pad(W2, ((0, 0), (0, 8 - _C)))
    b2p = jnp.pad(b2, (0, 8 - _C)).reshape(1, 8)

    degp = _deg_call(colp, zero_hist)
    hp = _mm_call(x_p, W1, degp)
    partials = _scat_call(hp, rowp, colp, zero_blk)
    out = _tail_call(partials, hp, degp, b1r, W2p, b2p)
    return out[:_N, :_C]


# speedup vs baseline: 1.1617x; 1.1617x over previous
import functools

import jax
import jax.numpy as jnp
from jax import lax
from jax.experimental import pallas as pl
from jax.experimental.pallas import tpu as pltpu
from jax.experimental.pallas import tpu_sc as plsc

_N = 10000
_E = 160000
_D = 256
_H = 128
_C = 2

_NP = 10240
_NC, _NS = 2, 16
_NW = _NC * _NS
_EP = 163840
_EPW = _EP // _NW
_CH = 128
_NCHUNK = _EPW // _CH
_RPS = _NP // _NS
_NBUF = 2
_HR = _NP // 2

_mesh = plsc.VectorSubcoreMesh(core_axis_name="c", subcore_axis_name="s")


def _deg_body(col_hbm, zero_hbm, out_hbm, colbuf, lhist, cntbuf):
    c = lax.axis_index("c")
    s = lax.axis_index("s")
    wid = c * _NS + s
    pltpu.sync_copy(col_hbm.at[pl.ds(wid * _EPW, _EPW)], colbuf)
    iota = lax.iota(jnp.int32, 16)
    laneoff = iota * _HR
    ones16 = jnp.ones((16,), jnp.float32)

    for p in range(_NP // _HR):
        pltpu.sync_copy(zero_hbm, lhist)

        def _scan(i, carry):
            idx = colbuf[pl.ds(i * 16, 16)]
            rel = idx - p * _HR
            m = (rel >= 0) & (rel < _HR)
            relc = jnp.where(m, rel, 0)
            addr = laneoff + relc
            cur = plsc.load_gather(lhist, [addr], mask=m)
            plsc.store_scatter(lhist, [addr], cur + ones16, mask=m)
            return carry

        lax.fori_loop(0, _EPW // 16, _scan, 0)

        def _reduce(k, carry):
            acc = lhist[pl.ds(k * 16, 16)]
            for t in range(1, 16):
                acc = acc + lhist[pl.ds(t * _HR + k * 16, 16)]
            cntbuf[pl.ds(k * 16, 16)] = acc
            return carry

        lax.fori_loop(0, _HR // 16, _reduce, 0)
        pltpu.sync_copy(cntbuf, out_hbm.at[wid, pl.ds(p * _HR, _HR)])


_deg_call = functools.partial(
    pl.kernel,
    out_type=jax.ShapeDtypeStruct((_NW, _NP), jnp.float32),
    mesh=_mesh,
    scratch_types=[
        pltpu.VMEM((_EPW,), jnp.int32),
        pltpu.VMEM((16 * _HR,), jnp.float32),
        pltpu.VMEM((_HR,), jnp.float32),
    ],
    compiler_params=pltpu.CompilerParams(needs_layout_passes=False),
)(_deg_body)


_CHF = 40
_CHS = 80 - _CHF


def _scat_body(hp_hbm, row2_hbm, col2_hbm, zero_hbm, out_hbm,
               rowbuf, colbuf, gbuf, accum,
               gs0, gs1, ss0, ss1):
    gsems = (gs0, gs1)
    ssems = (ss0, ss1)
    c = lax.axis_index("c")
    s = lax.axis_index("s")
    nch = jnp.where(c == 0, _CHF, _CHS)
    mybase = jnp.where(c == 0, s * _CHF, _NS * _CHF + s * _CHS)
    pltpu.sync_copy(zero_hbm, accum.at[pl.ds(s * _RPS, _RPS)])
    pltpu.sync_copy(row2_hbm.at[pl.ds(mybase, _CHF)], rowbuf)
    pltpu.sync_copy(col2_hbm.at[pl.ds(mybase, _CHF)], colbuf)
    plsc.subcore_barrier()

    def _start_g(j, b):
        pltpu.async_copy(hp_hbm.at[rowbuf.at[j]], gbuf.at[b], gsems[b])

    def _wait_g(j, b):
        pltpu.make_async_copy(hp_hbm.at[rowbuf.at[j]], gbuf.at[b],
                              gsems[b]).wait()

    def _chunk(j, carry):
        pltpu.async_copy(hp_hbm.at[rowbuf.at[j]], gbuf.at[0], gsems[0]).wait()
        pltpu.sync_copy(gbuf.at[0], accum.at[colbuf.at[j]], add=True)
        return carry

    lax.fori_loop(0, nch, _chunk, 0)

    plsc.subcore_barrier()
    pltpu.sync_copy(accum.at[pl.ds(s * _RPS, _RPS)],
                    out_hbm.at[c, pl.ds(s * _RPS, _RPS)])


_scat_call = functools.partial(
    pl.kernel,
    out_type=jax.ShapeDtypeStruct((_NC, _NP, _H), jnp.float32),
    mesh=_mesh,
    scratch_types=[
        pltpu.VMEM((_CHF, _CH), jnp.int32),
        pltpu.VMEM((_CHF, _CH), jnp.int32),
        pltpu.VMEM((_NBUF, _CH, _H), jnp.float32),
        pltpu.VMEM_SHARED((_NP, _H), jnp.float32),
        pltpu.SemaphoreType.DMA,
        pltpu.SemaphoreType.DMA,
        pltpu.SemaphoreType.DMA,
        pltpu.SemaphoreType.DMA,
    ],
)(_scat_body)


def _dis_block(degp_blk):
    deg = jnp.sum(degp_blk, axis=0) + 1.0
    dis = jax.lax.rsqrt(deg)
    return jax.lax.broadcast_in_dim(dis, (degp_blk.shape[1], _H), (0,))


def _mm_body(x_ref, w1_ref, degp_ref, hp_ref):
    dism = _dis_block(degp_ref[...])
    h = jnp.dot(x_ref[...], w1_ref[...], preferred_element_type=jnp.float32)
    hp_ref[...] = h * dism


_BM = 256


def _mm_call(x_p, W1, degp):
    grid = (_NP // _BM,)
    return pl.pallas_call(
        _mm_body,
        grid=grid,
        in_specs=[
            pl.BlockSpec((_BM, _D), lambda i: (i, 0)),
            pl.BlockSpec((_D, _H), lambda i: (0, 0)),
            pl.BlockSpec((_NW, _BM), lambda i: (0, i)),
        ],
        out_specs=pl.BlockSpec((_BM, _H), lambda i: (i, 0)),
        out_shape=jax.ShapeDtypeStruct((_NP, _H), jnp.float32),
    )(x_p, W1, degp)


def _tail_body(p_ref, hp_ref, degp_ref, b1_ref, w2_ref, b2_ref, out_ref):
    dism = _dis_block(degp_ref[...])
    sums = p_ref[0] + p_ref[1] + hp_ref[...]
    pre = sums * dism + b1_ref[...]
    act = jnp.maximum(pre, 0.0)
    out_ref[...] = jnp.dot(act, w2_ref[...],
                           preferred_element_type=jnp.float32) + b2_ref[...]


def _tail_call(partials, hp, degp, b1r, W2p, b2p):
    grid = (_NP // _BM,)
    return pl.pallas_call(
        _tail_body,
        grid=grid,
        in_specs=[
            pl.BlockSpec((_NC, _BM, _H), lambda i: (0, i, 0)),
            pl.BlockSpec((_BM, _H), lambda i: (i, 0)),
            pl.BlockSpec((_NW, _BM), lambda i: (0, i)),
            pl.BlockSpec((1, _H), lambda i: (0, 0)),
            pl.BlockSpec((_H, 8), lambda i: (0, 0)),
            pl.BlockSpec((1, 8), lambda i: (0, 0)),
        ],
        out_specs=pl.BlockSpec((_BM, 8), lambda i: (i, 0)),
        out_shape=jax.ShapeDtypeStruct((_NP, 8), jnp.float32),
    )(partials, hp, degp, b1r, W2p, b2p)


def kernel(x, edge_index, W1, b1, W2, b2):
    row = edge_index[0]
    col = edge_index[1]
    pad = _EP - _E
    rowp = jnp.concatenate([row, jnp.zeros((pad,), jnp.int32)])
    colp = jnp.concatenate([col, jnp.full((pad,), _N, jnp.int32)])
    row2 = rowp.reshape(_EP // _CH, _CH)
    col2 = colp.reshape(_EP // _CH, _CH)
    x_p = jnp.concatenate([x, jnp.zeros((_NP - _N, _D), jnp.float32)])
    zero_blk = jnp.zeros((_RPS, _H), jnp.float32)
    zero_hist = jnp.zeros((16 * _HR,), jnp.float32)
    b1r = b1.reshape(1, _H)
    W2p = jnp.pad(W2, ((0, 0), (0, 8 - _C)))
    b2p = jnp.pad(b2, (0, 8 - _C)).reshape(1, 8)

    degp = _deg_call(colp, zero_hist)
    hp = _mm_call(x_p, W1, degp)
    partials = _scat_call(hp, row2, col2, zero_blk)
    out = _tail_call(partials, hp, degp, b1r, W2p, b2p)
    return out[:_N, :_C]


# speedup vs baseline: 1.3347x; 1.1489x over previous
import functools

import jax
import jax.numpy as jnp
from jax import lax
from jax.experimental import pallas as pl
from jax.experimental.pallas import tpu as pltpu
from jax.experimental.pallas import tpu_sc as plsc

_N = 10000
_E = 160000
_D = 256
_H = 128
_C = 2

_NP = 10240
_NC, _NS = 2, 16
_NW = _NC * _NS
_EP = 163840
_EPW = _EP // _NW
_CH = 128
_NCHUNK = _EPW // _CH
_RPS = _NP // _NS

_mesh = plsc.VectorSubcoreMesh(core_axis_name="c", subcore_axis_name="s")


def _deg_body(col2_hbm, val_hbm, zero_hbm, out_hbm, colbuf, valbuf, degacc):
    c = lax.axis_index("c")
    s = lax.axis_index("s")
    wid = c * _NS + s
    pltpu.sync_copy(zero_hbm, degacc.at[pl.ds(s * _RPS, _RPS)])
    pltpu.sync_copy(val_hbm, valbuf)
    pltpu.sync_copy(col2_hbm.at[pl.ds(wid * _NCHUNK, _NCHUNK)], colbuf)
    plsc.subcore_barrier()

    def _chunk(j, carry):
        pltpu.sync_copy(valbuf, degacc.at[colbuf.at[j]], add=True)
        return carry

    lax.fori_loop(0, _NCHUNK, _chunk, 0)
    plsc.subcore_barrier()
    pltpu.sync_copy(degacc.at[pl.ds(s * _RPS, _RPS)],
                    out_hbm.at[c, pl.ds(s * _RPS, _RPS)])


_deg_call = functools.partial(
    pl.kernel,
    out_type=jax.ShapeDtypeStruct((_NC, _NP, _H), jnp.float32),
    mesh=_mesh,
    scratch_types=[
        pltpu.VMEM((_NCHUNK, _CH), jnp.int32),
        pltpu.VMEM((_CH, _H), jnp.float32),
        pltpu.VMEM_SHARED((_NP, _H), jnp.float32),
    ],
)(_deg_body)


def _scat_body(hp_hbm, row2_hbm, col2_hbm, zero_hbm, out_hbm,
               rowbuf, colbuf, gbuf, accum, sem):
    c = lax.axis_index("c")
    s = lax.axis_index("s")
    wid = c * _NS + s
    pltpu.sync_copy(zero_hbm, accum.at[pl.ds(s * _RPS, _RPS)])
    pltpu.sync_copy(row2_hbm.at[pl.ds(wid * _NCHUNK, _NCHUNK)], rowbuf)
    pltpu.sync_copy(col2_hbm.at[pl.ds(wid * _NCHUNK, _NCHUNK)], colbuf)
    plsc.subcore_barrier()

    def _chunk(j, carry):
        pltpu.async_copy(hp_hbm.at[rowbuf.at[j]], gbuf, sem).wait()
        pltpu.sync_copy(gbuf, accum.at[colbuf.at[j]], add=True)
        return carry

    lax.fori_loop(0, _NCHUNK, _chunk, 0)
    plsc.subcore_barrier()
    pltpu.sync_copy(accum.at[pl.ds(s * _RPS, _RPS)],
                    out_hbm.at[c, pl.ds(s * _RPS, _RPS)])


_scat_call = functools.partial(
    pl.kernel,
    out_type=jax.ShapeDtypeStruct((_NC, _NP, _H), jnp.float32),
    mesh=_mesh,
    scratch_types=[
        pltpu.VMEM((_NCHUNK, _CH), jnp.int32),
        pltpu.VMEM((_NCHUNK, _CH), jnp.int32),
        pltpu.VMEM((_CH, _H), jnp.float32),
        pltpu.VMEM_SHARED((_NP, _H), jnp.float32),
        pltpu.SemaphoreType.DMA,
    ],
)(_scat_body)


def _mm_body(x_ref, w1_ref, degt_ref, hp_ref):
    deg = jnp.sum(degt_ref[...], axis=1, keepdims=True) + 1.0
    dis = jax.lax.rsqrt(deg)
    h = jnp.dot(x_ref[...], w1_ref[...], preferred_element_type=jnp.float32)
    hp_ref[...] = h * dis


_BM = 256


def _mm_call(x_p, W1, degt):
    grid = (_NP // _BM,)
    return pl.pallas_call(
        _mm_body,
        grid=grid,
        in_specs=[
            pl.BlockSpec((_BM, _D), lambda i: (i, 0)),
            pl.BlockSpec((_D, _H), lambda i: (0, 0)),
            pl.BlockSpec((_BM, _NC * _H), lambda i: (i, 0)),
        ],
        out_specs=pl.BlockSpec((_BM, _H), lambda i: (i, 0)),
        out_shape=jax.ShapeDtypeStruct((_NP, _H), jnp.float32),
    )(x_p, W1, degt)


def _tail_body(p_ref, hp_ref, degt_ref, b1_ref, w2_ref, b2_ref, out_ref):
    deg = jnp.sum(degt_ref[...], axis=1, keepdims=True) + 1.0
    dis = jax.lax.rsqrt(deg)
    sums = p_ref[0] + p_ref[1] + hp_ref[...]
    pre = sums * dis + b1_ref[...]
    act = jnp.maximum(pre, 0.0)
    out_ref[...] = jnp.dot(act, w2_ref[...],
                           preferred_element_type=jnp.float32) + b2_ref[...]


def _tail_call(partials, hp, degt, b1r, W2p, b2p):
    grid = (_NP // _BM,)
    return pl.pallas_call(
        _tail_body,
        grid=grid,
        in_specs=[
            pl.BlockSpec((_NC, _BM, _H), lambda i: (0, i, 0)),
            pl.BlockSpec((_BM, _H), lambda i: (i, 0)),
            pl.BlockSpec((_BM, _NC * _H), lambda i: (i, 0)),
            pl.BlockSpec((1, _H), lambda i: (0, 0)),
            pl.BlockSpec((_H, 8), lambda i: (0, 0)),
            pl.BlockSpec((1, 8), lambda i: (0, 0)),
        ],
        out_specs=pl.BlockSpec((_BM, 8), lambda i: (i, 0)),
        out_shape=jax.ShapeDtypeStruct((_NP, 8), jnp.float32),
    )(partials, hp, degt, b1r, W2p, b2p)


def kernel(x, edge_index, W1, b1, W2, b2):
    row = edge_index[0]
    col = edge_index[1]
    pad = _EP - _E
    rowp = jnp.concatenate([row, jnp.zeros((pad,), jnp.int32)])
    colp = jnp.concatenate([col, jnp.full((pad,), _N, jnp.int32)])
    row2 = rowp.reshape(_EP // _CH, _CH)
    col2 = colp.reshape(_EP // _CH, _CH)
    x_p = jnp.concatenate([x, jnp.zeros((_NP - _N, _D), jnp.float32)])
    zero_blk = jnp.zeros((_RPS, _H), jnp.float32)
    val128 = jnp.full((_CH, _H), 1.0 / _H, jnp.float32)
    b1r = b1.reshape(1, _H)
    W2p = jnp.pad(W2, ((0, 0), (0, 8 - _C)))
    b2p = jnp.pad(b2, (0, 8 - _C)).reshape(1, 8)

    degp = _deg_call(col2, val128, zero_blk)
    degt = degp.transpose(1, 0, 2).reshape(_NP, _NC * _H)
    hp = _mm_call(x_p, W1, degt)
    partials = _scat_call(hp, row2, col2, zero_blk)
    out = _tail_call(partials, hp, degt, b1r, W2p, b2p)
    return out[:_N, :_C]


# speedup vs baseline: 1.3416x; 1.0052x over previous
import functools

import jax
import jax.numpy as jnp
from jax import lax
from jax.experimental import pallas as pl
from jax.experimental.pallas import tpu as pltpu
from jax.experimental.pallas import tpu_sc as plsc

_N = 10000
_E = 160000
_D = 256
_H = 128
_C = 2

_NP = 10240
_NC, _NS = 2, 16
_NW = _NC * _NS
_EP = 163840
_EPW = _EP // _NW
_CH = 128
_NCHUNK = _EPW // _CH
_RPS = _NP // _NS

_mesh = plsc.VectorSubcoreMesh(core_axis_name="c", subcore_axis_name="s")


def _deg_body(col2_hbm, val_hbm, zero_hbm, out_hbm, colbuf, valbuf, degacc):
    c = lax.axis_index("c")
    s = lax.axis_index("s")
    wid = c * _NS + s
    pltpu.sync_copy(zero_hbm, degacc.at[pl.ds(s * _RPS, _RPS)])
    pltpu.sync_copy(val_hbm, valbuf)
    pltpu.sync_copy(col2_hbm.at[pl.ds(wid * _NCHUNK, _NCHUNK)], colbuf)
    plsc.subcore_barrier()

    def _chunk(j, carry):
        pltpu.sync_copy(valbuf, degacc.at[colbuf.at[j]], add=True)
        return carry

    lax.fori_loop(0, _NCHUNK, _chunk, 0)
    plsc.subcore_barrier()
    pltpu.sync_copy(degacc.at[pl.ds(s * _RPS, _RPS)],
                    out_hbm.at[c, pl.ds(s * _RPS, _RPS)])


_deg_call = functools.partial(
    pl.kernel,
    out_type=jax.ShapeDtypeStruct((_NC, _NP, _H), jnp.float32),
    mesh=_mesh,
    scratch_types=[
        pltpu.VMEM((_NCHUNK, _CH), jnp.int32),
        pltpu.VMEM((_CH, _H), jnp.float32),
        pltpu.VMEM_SHARED((_NP, _H), jnp.float32),
    ],
)(_deg_body)


def _scat_body(hp_hbm, row2_hbm, col2_hbm, zero_hbm, out_hbm,
               rowbuf, colbuf, gbuf, accum, sem, sem2):
    c = lax.axis_index("c")
    s = lax.axis_index("s")
    wid = c * _NS + s
    pltpu.sync_copy(zero_hbm, accum.at[pl.ds(s * _RPS, _RPS)])
    pltpu.sync_copy(row2_hbm.at[pl.ds(wid * _NCHUNK, _NCHUNK)], rowbuf)
    pltpu.sync_copy(col2_hbm.at[pl.ds(wid * _NCHUNK, _NCHUNK)], colbuf)
    plsc.subcore_barrier()

    def _chunk(j, carry):
        h1 = pltpu.async_copy(hp_hbm.at[rowbuf.at[j, pl.ds(0, 64)]],
                              gbuf.at[pl.ds(0, 64)], sem)
        h2 = pltpu.async_copy(hp_hbm.at[rowbuf.at[j, pl.ds(64, 64)]],
                              gbuf.at[pl.ds(64, 64)], sem2)
        h1.wait()
        h2.wait()
        pltpu.sync_copy(gbuf, accum.at[colbuf.at[j]], add=True)
        return carry

    lax.fori_loop(0, _NCHUNK, _chunk, 0)
    plsc.subcore_barrier()
    pltpu.sync_copy(accum.at[pl.ds(s * _RPS, _RPS)],
                    out_hbm.at[c, pl.ds(s * _RPS, _RPS)])


_scat_call = functools.partial(
    pl.kernel,
    out_type=jax.ShapeDtypeStruct((_NC, _NP, _H), jnp.float32),
    mesh=_mesh,
    scratch_types=[
        pltpu.VMEM((_NCHUNK, _CH), jnp.int32),
        pltpu.VMEM((_NCHUNK, _CH), jnp.int32),
        pltpu.VMEM((_CH, _H), jnp.float32),
        pltpu.VMEM_SHARED((_NP, _H), jnp.float32),
        pltpu.SemaphoreType.DMA,
        pltpu.SemaphoreType.DMA,
    ],
)(_scat_body)


def _mm_body(x_ref, w1_ref, degt_ref, hp_ref):
    deg = jnp.sum(degt_ref[...], axis=1, keepdims=True) + 1.0
    dis = jax.lax.rsqrt(deg)
    h = jnp.dot(x_ref[...], w1_ref[...], preferred_element_type=jnp.float32)
    hp_ref[...] = h * dis


_BM = 256


def _mm_call(x_p, W1, degt):
    grid = (_NP // _BM,)
    return pl.pallas_call(
        _mm_body,
        grid=grid,
        in_specs=[
            pl.BlockSpec((_BM, _D), lambda i: (i, 0)),
            pl.BlockSpec((_D, _H), lambda i: (0, 0)),
            pl.BlockSpec((_BM, _NC * _H), lambda i: (i, 0)),
        ],
        out_specs=pl.BlockSpec((_BM, _H), lambda i: (i, 0)),
        out_shape=jax.ShapeDtypeStruct((_NP, _H), jnp.float32),
    )(x_p, W1, degt)


def _tail_body(p_ref, hp_ref, degt_ref, b1_ref, w2_ref, b2_ref, out_ref):
    deg = jnp.sum(degt_ref[...], axis=1, keepdims=True) + 1.0
    dis = jax.lax.rsqrt(deg)
    sums = p_ref[0] + p_ref[1] + hp_ref[...]
    pre = sums * dis + b1_ref[...]
    act = jnp.maximum(pre, 0.0)
    out_ref[...] = jnp.dot(act, w2_ref[...],
                           preferred_element_type=jnp.float32) + b2_ref[...]


def _tail_call(partials, hp, degt, b1r, W2p, b2p):
    grid = (_NP // _BM,)
    return pl.pallas_call(
        _tail_body,
        grid=grid,
        in_specs=[
            pl.BlockSpec((_NC, _BM, _H), lambda i: (0, i, 0)),
            pl.BlockSpec((_BM, _H), lambda i: (i, 0)),
            pl.BlockSpec((_BM, _NC * _H), lambda i: (i, 0)),
            pl.BlockSpec((1, _H), lambda i: (0, 0)),
            pl.BlockSpec((_H, 8), lambda i: (0, 0)),
            pl.BlockSpec((1, 8), lambda i: (0, 0)),
        ],
        out_specs=pl.BlockSpec((_BM, 8), lambda i: (i, 0)),
        out_shape=jax.ShapeDtypeStruct((_NP, 8), jnp.float32),
    )(partials, hp, degt, b1r, W2p, b2p)


def kernel(x, edge_index, W1, b1, W2, b2):
    row = edge_index[0]
    col = edge_index[1]
    pad = _EP - _E
    rowp = jnp.concatenate([row, jnp.zeros((pad,), jnp.int32)])
    colp = jnp.concatenate([col, jnp.full((pad,), _N, jnp.int32)])
    row2 = rowp.reshape(_EP // _CH, _CH)
    col2 = colp.reshape(_EP // _CH, _CH)
    x_p = jnp.concatenate([x, jnp.zeros((_NP - _N, _D), jnp.float32)])
    zero_blk = jnp.zeros((_RPS, _H), jnp.float32)
    val128 = jnp.full((_CH, _H), 1.0 / _H, jnp.float32)
    b1r = b1.reshape(1, _H)
    W2p = jnp.pad(W2, ((0, 0), (0, 8 - _C)))
    b2p = jnp.pad(b2, (0, 8 - _C)).reshape(1, 8)

    degp = _deg_call(col2, val128, zero_blk)
    degt = degp.transpose(1, 0, 2).reshape(_NP, _NC * _H)
    hp = _mm_call(x_p, W1, degt)
    partials = _scat_call(hp, row2, col2, zero_blk)
    out = _tail_call(partials, hp, degt, b1r, W2p, b2p)
    return out[:_N, :_C]


# speedup vs baseline: 1.3498x; 1.0061x over previous
import functools

import jax
import jax.numpy as jnp
from jax import lax
from jax.experimental import pallas as pl
from jax.experimental.pallas import tpu as pltpu
from jax.experimental.pallas import tpu_sc as plsc

_N = 10000
_E = 160000
_D = 256
_H = 128
_C = 2

_NP = 10240
_NC, _NS = 2, 16
_NW = _NC * _NS
_EP = 163840
_EPW = _EP // _NW
_CH = 128
_NCHUNK = _EPW // _CH
_RPS = _NP // _NS

_mesh = plsc.VectorSubcoreMesh(core_axis_name="c", subcore_axis_name="s")


def _deg_body(col2_hbm, val_hbm, zero_hbm, out_hbm, colbuf, valbuf, degacc):
    c = lax.axis_index("c")
    s = lax.axis_index("s")
    wid = c * _NS + s
    pltpu.sync_copy(zero_hbm, degacc.at[pl.ds(s * _RPS, _RPS)])
    pltpu.sync_copy(val_hbm, valbuf)
    pltpu.sync_copy(col2_hbm.at[pl.ds(wid * _NCHUNK, _NCHUNK)], colbuf)
    plsc.subcore_barrier()

    def _chunk(j, carry):
        pltpu.sync_copy(valbuf, degacc.at[colbuf.at[j]], add=True)
        return carry

    lax.fori_loop(0, _NCHUNK, _chunk, 0)
    plsc.subcore_barrier()
    pltpu.sync_copy(degacc.at[pl.ds(s * _RPS, _RPS)],
                    out_hbm.at[c, pl.ds(s * _RPS, _RPS)])


_deg_call = functools.partial(
    pl.kernel,
    out_type=jax.ShapeDtypeStruct((_NC, _NP, _H), jnp.float32),
    mesh=_mesh,
    scratch_types=[
        pltpu.VMEM((_NCHUNK, _CH), jnp.int32),
        pltpu.VMEM((_CH, _H), jnp.float32),
        pltpu.VMEM_SHARED((_NP, _H), jnp.float32),
    ],
)(_deg_body)


def _scat_body(hp_hbm, row2_hbm, col2_hbm, zero_hbm, out_hbm,
               rowbuf, colbuf, gbuf, accum, sem, sem2):
    c = lax.axis_index("c")
    s = lax.axis_index("s")
    wid = c * _NS + s
    pltpu.sync_copy(zero_hbm, accum.at[pl.ds(s * _RPS, _RPS)])
    pltpu.sync_copy(row2_hbm.at[pl.ds(wid * _NCHUNK, _NCHUNK)], rowbuf)
    pltpu.sync_copy(col2_hbm.at[pl.ds(wid * _NCHUNK, _NCHUNK)], colbuf)
    plsc.subcore_barrier()

    def _chunk(j, carry):
        h1 = pltpu.async_copy(hp_hbm.at[rowbuf.at[j, pl.ds(0, 64)]],
                              gbuf.at[pl.ds(0, 64)], sem)
        h2 = pltpu.async_copy(hp_hbm.at[rowbuf.at[j, pl.ds(64, 64)]],
                              gbuf.at[pl.ds(64, 64)], sem2)
        h1.wait()
        h2.wait()
        pltpu.sync_copy(gbuf, accum.at[colbuf.at[j]], add=True)
        return carry

    lax.fori_loop(0, _NCHUNK, _chunk, 0)
    plsc.subcore_barrier()
    pltpu.sync_copy(accum.at[pl.ds(s * _RPS, _RPS)],
                    out_hbm.at[c, pl.ds(s * _RPS, _RPS)])


_scat_call = functools.partial(
    pl.kernel,
    out_type=jax.ShapeDtypeStruct((_NC, _NP, _H), jnp.float32),
    mesh=_mesh,
    scratch_types=[
        pltpu.VMEM((_NCHUNK, _CH), jnp.int32),
        pltpu.VMEM((_NCHUNK, _CH), jnp.int32),
        pltpu.VMEM((_CH, _H), jnp.float32),
        pltpu.VMEM_SHARED((_NP, _H), jnp.float32),
        pltpu.SemaphoreType.DMA,
        pltpu.SemaphoreType.DMA,
    ],
)(_scat_body)


def _mm_body(x_ref, w1_ref, h_ref):
    h_ref[...] = jnp.dot(x_ref[...], w1_ref[...],
                         preferred_element_type=jnp.float32)


_BM = 256


def _mm_call(x_p, W1):
    grid = (_NP // _BM,)
    return pl.pallas_call(
        _mm_body,
        grid=grid,
        in_specs=[
            pl.BlockSpec((_BM, _D), lambda i: (i, 0)),
            pl.BlockSpec((_D, _H), lambda i: (0, 0)),
        ],
        out_specs=pl.BlockSpec((_BM, _H), lambda i: (i, 0)),
        out_shape=jax.ShapeDtypeStruct((_NP, _H), jnp.float32),
    )(x_p, W1)


def _scale_body(h_ref, degt_ref, hp_ref):
    deg = jnp.sum(degt_ref[...], axis=1, keepdims=True) + 1.0
    dis = jax.lax.rsqrt(deg)
    hp_ref[...] = h_ref[...] * dis


def _scale_call(h, degt):
    grid = (_NP // _BM,)
    return pl.pallas_call(
        _scale_body,
        grid=grid,
        in_specs=[
            pl.BlockSpec((_BM, _H), lambda i: (i, 0)),
            pl.BlockSpec((_BM, _NC * _H), lambda i: (i, 0)),
        ],
        out_specs=pl.BlockSpec((_BM, _H), lambda i: (i, 0)),
        out_shape=jax.ShapeDtypeStruct((_NP, _H), jnp.float32),
    )(h, degt)


def _tail_body(p_ref, hp_ref, degt_ref, b1_ref, w2_ref, b2_ref, out_ref):
    deg = jnp.sum(degt_ref[...], axis=1, keepdims=True) + 1.0
    dis = jax.lax.rsqrt(deg)
    sums = p_ref[0] + p_ref[1] + hp_ref[...]
    pre = sums * dis + b1_ref[...]
    act = jnp.maximum(pre, 0.0)
    out_ref[...] = jnp.dot(act, w2_ref[...],
                           preferred_element_type=jnp.float32) + b2_ref[...]


def _tail_call(partials, hp, degt, b1r, W2p, b2p):
    grid = (_NP // _BM,)
    return pl.pallas_call(
        _tail_body,
        grid=grid,
        in_specs=[
            pl.BlockSpec((_NC, _BM, _H), lambda i: (0, i, 0)),
            pl.BlockSpec((_BM, _H), lambda i: (i, 0)),
            pl.BlockSpec((_BM, _NC * _H), lambda i: (i, 0)),
            pl.BlockSpec((1, _H), lambda i: (0, 0)),
            pl.BlockSpec((_H, 8), lambda i: (0, 0)),
            pl.BlockSpec((1, 8), lambda i: (0, 0)),
        ],
        out_specs=pl.BlockSpec((_BM, 8), lambda i: (i, 0)),
        out_shape=jax.ShapeDtypeStruct((_NP, 8), jnp.float32),
    )(partials, hp, degt, b1r, W2p, b2p)


def kernel(x, edge_index, W1, b1, W2, b2):
    row = edge_index[0]
    col = edge_index[1]
    pad = _EP - _E
    rowp = jnp.concatenate([row, jnp.zeros((pad,), jnp.int32)])
    colp = jnp.concatenate([col, jnp.full((pad,), _N, jnp.int32)])
    row2 = rowp.reshape(_EP // _CH, _CH)
    col2 = colp.reshape(_EP // _CH, _CH)
    x_p = jnp.concatenate([x, jnp.zeros((_NP - _N, _D), jnp.float32)])
    zero_blk = jnp.zeros((_RPS, _H), jnp.float32)
    val128 = jnp.full((_CH, _H), 1.0 / _H, jnp.float32)
    b1r = b1.reshape(1, _H)
    W2p = jnp.pad(W2, ((0, 0), (0, 8 - _C)))
    b2p = jnp.pad(b2, (0, 8 - _C)).reshape(1, 8)

    degp = _deg_call(col2, val128, zero_blk)
    h = _mm_call(x_p, W1)
    degt = degp.transpose(1, 0, 2).reshape(_NP, _NC * _H)
    hp = _scale_call(h, degt)
    partials = _scat_call(hp, row2, col2, zero_blk)
    out = _tail_call(partials, hp, degt, b1r, W2p, b2p)
    return out[:_N, :_C]


# speedup vs baseline: 1.3515x; 1.0012x over previous
import functools

import jax
import jax.numpy as jnp
from jax import lax
from jax.experimental import pallas as pl
from jax.experimental.pallas import tpu as pltpu
from jax.experimental.pallas import tpu_sc as plsc

_N = 10000
_E = 160000
_D = 256
_H = 128
_C = 2

_NP = 10240
_NC, _NS = 2, 16
_NW = _NC * _NS
_EP = 163840
_EPW = _EP // _NW
_CH = 128
_NCHUNK = _EPW // _CH
_RPS = _NP // _NS

_mesh = plsc.VectorSubcoreMesh(core_axis_name="c", subcore_axis_name="s")


def _deg_body(col2_hbm, val_hbm, zero_hbm, out_hbm, colbuf, valbuf, degacc, dsem):
    c = lax.axis_index("c")
    s = lax.axis_index("s")
    wid = c * _NS + s
    pltpu.sync_copy(zero_hbm, degacc.at[pl.ds(s * _RPS, _RPS)])
    pltpu.sync_copy(val_hbm, valbuf)
    pltpu.sync_copy(col2_hbm.at[pl.ds(wid * _NCHUNK, _NCHUNK)], colbuf)
    plsc.subcore_barrier()

    def _fire(j, carry):
        pltpu.async_copy(valbuf, degacc.at[colbuf.at[j]], dsem, add=True)
        return carry

    lax.fori_loop(0, _NCHUNK, _fire, 0)

    def _drain(j, carry):
        pltpu.make_async_copy(valbuf, degacc.at[colbuf.at[j]], dsem).wait()
        return carry

    lax.fori_loop(0, _NCHUNK, _drain, 0)
    plsc.subcore_barrier()
    pltpu.sync_copy(degacc.at[pl.ds(s * _RPS, _RPS)],
                    out_hbm.at[c, pl.ds(s * _RPS, _RPS)])


_deg_call = functools.partial(
    pl.kernel,
    out_type=jax.ShapeDtypeStruct((_NC, _NP, _H), jnp.float32),
    mesh=_mesh,
    scratch_types=[
        pltpu.VMEM((_NCHUNK, _CH), jnp.int32),
        pltpu.VMEM((_CH, _H), jnp.float32),
        pltpu.VMEM_SHARED((_NP, _H), jnp.float32),
        pltpu.SemaphoreType.DMA,
    ],
)(_deg_body)


def _scat_body(hp_hbm, row2_hbm, col2_hbm, zero_hbm, out_hbm,
               rowbuf, colbuf, gbuf, accum, sem, sem2):
    c = lax.axis_index("c")
    s = lax.axis_index("s")
    wid = c * _NS + s
    pltpu.sync_copy(zero_hbm, accum.at[pl.ds(s * _RPS, _RPS)])
    pltpu.sync_copy(row2_hbm.at[pl.ds(wid * _NCHUNK, _NCHUNK)], rowbuf)
    pltpu.sync_copy(col2_hbm.at[pl.ds(wid * _NCHUNK, _NCHUNK)], colbuf)
    plsc.subcore_barrier()

    def _chunk(j, carry):
        h1 = pltpu.async_copy(hp_hbm.at[rowbuf.at[j, pl.ds(0, 64)]],
                              gbuf.at[pl.ds(0, 64)], sem)
        h2 = pltpu.async_copy(hp_hbm.at[rowbuf.at[j, pl.ds(64, 64)]],
                              gbuf.at[pl.ds(64, 64)], sem2)
        h1.wait()
        h2.wait()
        pltpu.sync_copy(gbuf, accum.at[colbuf.at[j]], add=True)
        return carry

    lax.fori_loop(0, _NCHUNK, _chunk, 0)
    plsc.subcore_barrier()
    pltpu.sync_copy(accum.at[pl.ds(s * _RPS, _RPS)],
                    out_hbm.at[c, pl.ds(s * _RPS, _RPS)])


_scat_call = functools.partial(
    pl.kernel,
    out_type=jax.ShapeDtypeStruct((_NC, _NP, _H), jnp.float32),
    mesh=_mesh,
    scratch_types=[
        pltpu.VMEM((_NCHUNK, _CH), jnp.int32),
        pltpu.VMEM((_NCHUNK, _CH), jnp.int32),
        pltpu.VMEM((_CH, _H), jnp.float32),
        pltpu.VMEM_SHARED((_NP, _H), jnp.float32),
        pltpu.SemaphoreType.DMA,
        pltpu.SemaphoreType.DMA,
    ],
)(_scat_body)


def _mm_body(x_ref, w1_ref, h_ref):
    h_ref[...] = jnp.dot(x_ref[...], w1_ref[...],
                         preferred_element_type=jnp.float32)


_BM = 256


def _mm_call(x_p, W1):
    grid = (_NP // _BM,)
    return pl.pallas_call(
        _mm_body,
        grid=grid,
        in_specs=[
            pl.BlockSpec((_BM, _D), lambda i: (i, 0)),
            pl.BlockSpec((_D, _H), lambda i: (0, 0)),
        ],
        out_specs=pl.BlockSpec((_BM, _H), lambda i: (i, 0)),
        out_shape=jax.ShapeDtypeStruct((_NP, _H), jnp.float32),
    )(x_p, W1)


def _scale_body(h_ref, degt_ref, hp_ref):
    deg = jnp.sum(degt_ref[...], axis=1, keepdims=True) + 1.0
    dis = jax.lax.rsqrt(deg)
    hp_ref[...] = h_ref[...] * dis


def _scale_call(h, degt):
    grid = (_NP // _BM,)
    return pl.pallas_call(
        _scale_body,
        grid=grid,
        in_specs=[
            pl.BlockSpec((_BM, _H), lambda i: (i, 0)),
            pl.BlockSpec((_BM, _NC * _H), lambda i: (i, 0)),
        ],
        out_specs=pl.BlockSpec((_BM, _H), lambda i: (i, 0)),
        out_shape=jax.ShapeDtypeStruct((_NP, _H), jnp.float32),
    )(h, degt)


def _tail_body(p_ref, hp_ref, degt_ref, b1_ref, w2_ref, b2_ref, out_ref):
    deg = jnp.sum(degt_ref[...], axis=1, keepdims=True) + 1.0
    dis = jax.lax.rsqrt(deg)
    sums = p_ref[0] + p_ref[1] + hp_ref[...]
    pre = sums * dis + b1_ref[...]
    act = jnp.maximum(pre, 0.0)
    out_ref[...] = jnp.dot(act, w2_ref[...],
                           preferred_element_type=jnp.float32) + b2_ref[...]


def _tail_call(partials, hp, degt, b1r, W2p, b2p):
    grid = (_NP // _BM,)
    return pl.pallas_call(
        _tail_body,
        grid=grid,
        in_specs=[
            pl.BlockSpec((_NC, _BM, _H), lambda i: (0, i, 0)),
            pl.BlockSpec((_BM, _H), lambda i: (i, 0)),
            pl.BlockSpec((_BM, _NC * _H), lambda i: (i, 0)),
            pl.BlockSpec((1, _H), lambda i: (0, 0)),
            pl.BlockSpec((_H, 8), lambda i: (0, 0)),
            pl.BlockSpec((1, 8), lambda i: (0, 0)),
        ],
        out_specs=pl.BlockSpec((_BM, 8), lambda i: (i, 0)),
        out_shape=jax.ShapeDtypeStruct((_NP, 8), jnp.float32),
    )(partials, hp, degt, b1r, W2p, b2p)


def kernel(x, edge_index, W1, b1, W2, b2):
    row = edge_index[0]
    col = edge_index[1]
    pad = _EP - _E
    rowp = jnp.concatenate([row, jnp.zeros((pad,), jnp.int32)])
    colp = jnp.concatenate([col, jnp.full((pad,), _N, jnp.int32)])
    row2 = rowp.reshape(_EP // _CH, _CH)
    col2 = colp.reshape(_EP // _CH, _CH)
    x_p = jnp.concatenate([x, jnp.zeros((_NP - _N, _D), jnp.float32)])
    zero_blk = jnp.zeros((_RPS, _H), jnp.float32)
    val128 = jnp.full((_CH, _H), 1.0 / _H, jnp.float32)
    b1r = b1.reshape(1, _H)
    W2p = jnp.pad(W2, ((0, 0), (0, 8 - _C)))
    b2p = jnp.pad(b2, (0, 8 - _C)).reshape(1, 8)

    degp = _deg_call(col2, val128, zero_blk)
    h = _mm_call(x_p, W1)
    degt = degp.transpose(1, 0, 2).reshape(_NP, _NC * _H)
    hp = _scale_call(h, degt)
    partials = _scat_call(hp, row2, col2, zero_blk)
    out = _tail_call(partials, hp, degt, b1r, W2p, b2p)
    return out[:_N, :_C]


# speedup vs baseline: 1.4001x; 1.0360x over previous
import functools

import jax
import jax.numpy as jnp
from jax import lax
from jax.experimental import pallas as pl
from jax.experimental.pallas import tpu as pltpu
from jax.experimental.pallas import tpu_sc as plsc

_N = 10000
_E = 160000
_D = 256
_H = 128
_C = 2

_NP = 10240
_NC, _NS = 2, 16
_NW = _NC * _NS
_EP = 163840
_EPW = _EP // _NW
_CH = 128
_NCHUNK = _EPW // _CH
_RPS = _NP // _NS

_mesh = plsc.VectorSubcoreMesh(core_axis_name="c", subcore_axis_name="s")


def _deg_body(col2_hbm, val_hbm, zero_hbm, out_hbm, colbuf, valbuf, degacc, dsem):
    c = lax.axis_index("c")
    s = lax.axis_index("s")
    wid = c * _NS + s
    pltpu.sync_copy(zero_hbm, degacc.at[pl.ds(s * _RPS, _RPS)])
    pltpu.sync_copy(val_hbm, valbuf)
    pltpu.sync_copy(col2_hbm.at[pl.ds(wid * _NCHUNK, _NCHUNK)], colbuf)
    plsc.subcore_barrier()

    def _fire(j, carry):
        pltpu.async_copy(valbuf, degacc.at[colbuf.at[j]], dsem, add=True)
        return carry

    lax.fori_loop(0, _NCHUNK, _fire, 0)

    def _drain(j, carry):
        pltpu.make_async_copy(valbuf, degacc.at[colbuf.at[j]], dsem).wait()
        return carry

    lax.fori_loop(0, _NCHUNK, _drain, 0)
    plsc.subcore_barrier()
    pltpu.sync_copy(degacc.at[pl.ds(s * _RPS, _RPS)],
                    out_hbm.at[c, pl.ds(s * _RPS, _RPS)])


_deg_call = functools.partial(
    pl.kernel,
    out_type=jax.ShapeDtypeStruct((_NC, _NP, _H), jnp.float32),
    mesh=_mesh,
    scratch_types=[
        pltpu.VMEM((_NCHUNK, _CH), jnp.int32),
        pltpu.VMEM((_CH, _H), jnp.float32),
        pltpu.VMEM_SHARED((_NP, _H), jnp.float32),
        pltpu.SemaphoreType.DMA,
    ],
)(_deg_body)


def _scat_body(hp_hbm, row2_hbm, col2_hbm, zero_hbm, out_hbm,
               rowbuf, colbuf, gbuf, accum, sem, sem2):
    c = lax.axis_index("c")
    s = lax.axis_index("s")
    wid = c * _NS + s
    pltpu.sync_copy(zero_hbm, accum.at[pl.ds(s * _RPS, _RPS)])
    pltpu.sync_copy(row2_hbm.at[pl.ds(wid * _NCHUNK, _NCHUNK)], rowbuf)
    pltpu.sync_copy(col2_hbm.at[pl.ds(wid * _NCHUNK, _NCHUNK)], colbuf)
    plsc.subcore_barrier()

    def _chunk(j, carry):
        h1 = pltpu.async_copy(hp_hbm.at[rowbuf.at[j, pl.ds(0, 64)]],
                              gbuf.at[pl.ds(0, 64)], sem)
        h2 = pltpu.async_copy(hp_hbm.at[rowbuf.at[j, pl.ds(64, 64)]],
                              gbuf.at[pl.ds(64, 64)], sem2)
        h1.wait()
        h2.wait()
        pltpu.sync_copy(gbuf, accum.at[colbuf.at[j]], add=True)
        return carry

    lax.fori_loop(0, _NCHUNK, _chunk, 0)
    plsc.subcore_barrier()
    pltpu.sync_copy(accum.at[pl.ds(s * _RPS, _RPS)],
                    out_hbm.at[c, pl.ds(s * _RPS, _RPS)])


_scat_call = functools.partial(
    pl.kernel,
    out_type=jax.ShapeDtypeStruct((_NC, _NP, _H), jnp.float32),
    mesh=_mesh,
    scratch_types=[
        pltpu.VMEM((_NCHUNK, _CH), jnp.int32),
        pltpu.VMEM((_NCHUNK, _CH), jnp.int32),
        pltpu.VMEM((_CH, _H), jnp.float32),
        pltpu.VMEM_SHARED((_NP, _H), jnp.float32),
        pltpu.SemaphoreType.DMA,
        pltpu.SemaphoreType.DMA,
    ],
)(_scat_body)


def _mm_body(x_ref, w1_ref, h_ref):
    h_ref[...] = jnp.dot(x_ref[...], w1_ref[...],
                         preferred_element_type=jnp.float32)


_BM = 256


def _mm_call(x_p, W1):
    grid = (_NP // _BM,)
    return pl.pallas_call(
        _mm_body,
        grid=grid,
        in_specs=[
            pl.BlockSpec((_BM, _D), lambda i: (i, 0)),
            pl.BlockSpec((_D, _H), lambda i: (0, 0)),
        ],
        out_specs=pl.BlockSpec((_BM, _H), lambda i: (i, 0)),
        out_shape=jax.ShapeDtypeStruct((_NP, _H), jnp.float32),
    )(x_p, W1)


def _deg_from_partials(degp_blk):
    d = degp_blk[0] + degp_blk[1]
    return jnp.sum(d, axis=1, keepdims=True) + 1.0


def _scale_body(h_ref, degp_ref, hp_ref):
    dis = jax.lax.rsqrt(_deg_from_partials(degp_ref[...]))
    hp_ref[...] = h_ref[...] * dis


def _scale_call(h, degp):
    grid = (_NP // _BM,)
    return pl.pallas_call(
        _scale_body,
        grid=grid,
        in_specs=[
            pl.BlockSpec((_BM, _H), lambda i: (i, 0)),
            pl.BlockSpec((_NC, _BM, _H), lambda i: (0, i, 0)),
        ],
        out_specs=pl.BlockSpec((_BM, _H), lambda i: (i, 0)),
        out_shape=jax.ShapeDtypeStruct((_NP, _H), jnp.float32),
    )(h, degp)


def _tail_body(p_ref, hp_ref, degp_ref, b1_ref, w2_ref, b2_ref, out_ref):
    dis = jax.lax.rsqrt(_deg_from_partials(degp_ref[...]))
    sums = p_ref[0] + p_ref[1] + hp_ref[...]
    pre = sums * dis + b1_ref[...]
    act = jnp.maximum(pre, 0.0)
    out_ref[...] = jnp.dot(act, w2_ref[...],
                           preferred_element_type=jnp.float32) + b2_ref[...]


def _tail_call(partials, hp, degp, b1r, W2p, b2p):
    grid = (_NP // _BM,)
    return pl.pallas_call(
        _tail_body,
        grid=grid,
        in_specs=[
            pl.BlockSpec((_NC, _BM, _H), lambda i: (0, i, 0)),
            pl.BlockSpec((_BM, _H), lambda i: (i, 0)),
            pl.BlockSpec((_NC, _BM, _H), lambda i: (0, i, 0)),
            pl.BlockSpec((1, _H), lambda i: (0, 0)),
            pl.BlockSpec((_H, 8), lambda i: (0, 0)),
            pl.BlockSpec((1, 8), lambda i: (0, 0)),
        ],
        out_specs=pl.BlockSpec((_BM, 8), lambda i: (i, 0)),
        out_shape=jax.ShapeDtypeStruct((_NP, 8), jnp.float32),
    )(partials, hp, degp, b1r, W2p, b2p)


def kernel(x, edge_index, W1, b1, W2, b2):
    row = edge_index[0]
    col = edge_index[1]
    pad = _EP - _E
    rowp = jnp.concatenate([row, jnp.zeros((pad,), jnp.int32)])
    colp = jnp.concatenate([col, jnp.full((pad,), _N, jnp.int32)])
    row2 = rowp.reshape(_EP // _CH, _CH)
    col2 = colp.reshape(_EP // _CH, _CH)
    x_p = jnp.concatenate([x, jnp.zeros((_NP - _N, _D), jnp.float32)])
    zero_blk = jnp.zeros((_RPS, _H), jnp.float32)
    val128 = jnp.full((_CH, _H), 1.0 / _H, jnp.float32)
    b1r = b1.reshape(1, _H)
    W2p = jnp.pad(W2, ((0, 0), (0, 8 - _C)))
    b2p = jnp.pad(b2, (0, 8 - _C)).reshape(1, 8)

    degp = _deg_call(col2, val128, zero_blk)
    h = _mm_call(x_p, W1)
    hp = _scale_call(h, degp)
    partials = _scat_call(hp, row2, col2, zero_blk)
    out = _tail_call(partials, hp, degp, b1r, W2p, b2p)
    return out[:_N, :_C]
